# Initial kernel scaffold; baseline (speedup 1.0000x reference)
import jax
import jax.numpy as jnp
from jax.experimental import pallas as pl


def kernel(x, wg):
    raise NotImplementedError("write your pallas kernel here")


# speedup vs baseline: 1.4841x; 1.4841x over previous
import functools

import jax
import jax.numpy as jnp
from jax import lax
from jax.experimental import pallas as pl
from jax.experimental.pallas import tpu as pltpu
from jax.experimental.pallas import tpu_sc as plsc

T = 4096
D = 2048
E = 64
CAP = 64
ECAP = E * CAP
THR = 0.4
BT = 256
NB = T // BT

NW = 32
TPW = T // NW
GRP = 16
NGRP = TPW // GRP


def _route_kernel(x_ref, wg_ref, meta_ref, acc_ref):
    i = pl.program_id(0)

    @pl.when(i == 0)
    def _():
        acc_ref[...] = jnp.zeros_like(acc_ref)

    xb = x_ref[...]
    logits = jnp.dot(xb, wg_ref[...], preferred_element_type=jnp.float32)
    mx = jnp.max(logits, axis=1, keepdims=True)
    ex = jnp.exp(logits - mx)
    gates = ex / jnp.sum(ex, axis=1, keepdims=True)

    iota_e = jax.lax.broadcasted_iota(jnp.int32, (BT, E), 1).astype(jnp.float32)
    g0 = jnp.max(gates, axis=1, keepdims=True)
    e0 = jnp.min(jnp.where(gates == g0, iota_e, jnp.float32(E)),
                 axis=1, keepdims=True)
    oh0 = (iota_e == e0).astype(jnp.float32)

    gates1 = jnp.where(iota_e == e0, -jnp.inf, gates)
    g1 = jnp.max(gates1, axis=1, keepdims=True)
    e1 = jnp.min(jnp.where(gates1 == g1, iota_e, jnp.float32(E)),
                 axis=1, keepdims=True)
    flag1 = g0 < THR
    oh1 = ((iota_e == e1) & flag1).astype(jnp.float32)

    rr = jax.lax.broadcasted_iota(jnp.int32, (BT, BT), 0)
    cc = jax.lax.broadcasted_iota(jnp.int32, (BT, BT), 1)
    tril = (rr > cc).astype(jnp.float32)
    pb0 = jnp.dot(tril, oh0, preferred_element_type=jnp.float32)
    pb1 = jnp.dot(tril, oh1, preferred_element_type=jnp.float32)

    cnt0 = acc_ref[0:1, :]
    cnt1 = acc_ref[1:2, :]
    pos0 = jnp.sum((pb0 + cnt0) * oh0, axis=1, keepdims=True)
    c1v = jnp.sum((pb1 + cnt1) * oh1, axis=1, keepdims=True)
    c1 = jnp.where(flag1, c1v, jnp.float32(1e6))

    acc_ref[0:1, :] = cnt0 + jnp.sum(oh0, axis=0, keepdims=True)
    acc_ref[1:2, :] = cnt1 + jnp.sum(oh1, axis=0, keepdims=True)
    acc_ref[2:3, :] = acc_ref[2:3, :] + jnp.sum(gates, axis=0, keepdims=True)

    meta_ref[0] = jnp.concatenate(
        [e0, e1, g0, g1, pos0, c1, jnp.zeros((BT, 2), jnp.float32)], axis=1)

    @pl.when(i == NB - 1)
    def _():
        laux = jnp.sum(acc_ref[0:1, :] * acc_ref[2:3, :]) * (E / (T * T))
        acc_ref[3:4, :] = jnp.full((1, E), laux, jnp.float32)


def _mask_kernel(meta_ref, acc_ref, dm_ref):
    meta = meta_ref[0]
    e0 = meta[:, 0:1]
    e1 = meta[:, 1:2]
    pos0 = meta[:, 4:5]
    c1 = meta[:, 5:6]
    g0 = meta[:, 2:3]
    g1 = meta[:, 3:4]
    cnt0 = acc_ref[0:1, :]

    iota_e = jax.lax.broadcasted_iota(jnp.int32, (BT, E), 1).astype(jnp.float32)
    ohe1 = (iota_e == e1).astype(jnp.float32)
    pos1 = c1 + jnp.sum(cnt0 * ohe1, axis=1, keepdims=True)

    q0 = jnp.where((pos0 < CAP) & (g0 != 0.0),
                   e0 * CAP + pos0, jnp.float32(-1.0))
    q1 = jnp.where((pos1 < CAP) & (g1 != 0.0),
                   e1 * CAP + pos1, jnp.float32(-1.0))

    qi = jax.lax.broadcasted_iota(jnp.int32, (BT, ECAP), 1).astype(jnp.float32)
    dm_ref[...] = ((qi == q0) | (qi == q1)).astype(jnp.int8)


def _sc_cw_body(meta_hbm, acc_hbm, out_hbm, meta_v, cnt0_v, rows_v):
    c = lax.axis_index("c")
    s = lax.axis_index("s")
    w = s * 2 + c

    blk = w // 2
    off = (w % 2) * TPW
    pltpu.sync_copy(meta_hbm.at[blk, pl.ds(off, TPW), :], meta_v)
    pltpu.sync_copy(acc_hbm.at[0], cnt0_v)

    def _zrow(k, _):
        b = k * 64
        z = jnp.zeros((16,), jnp.float32)
        rows_v[pl.ds(b, 16)] = z
        rows_v[pl.ds(b + 16, 16)] = z
        rows_v[pl.ds(b + 32, 16)] = z
        rows_v[pl.ds(b + 48, 16)] = z
        return 0
    lax.fori_loop(0, GRP * ECAP // 64, _zrow, 0)

    zeros16 = jnp.zeros((16,), jnp.float32)
    rid = lax.iota(jnp.int32, 16)

    for g in range(NGRP):
        tids = rid + g * GRP
        f0 = jnp.full((16,), 0, jnp.int32)
        e0 = plsc.load_gather(meta_v, [tids, f0])
        e1 = plsc.load_gather(meta_v, [tids, f0 + 1])
        g0 = plsc.load_gather(meta_v, [tids, f0 + 2])
        g1 = plsc.load_gather(meta_v, [tids, f0 + 3])
        pos0 = plsc.load_gather(meta_v, [tids, f0 + 4])
        c1 = plsc.load_gather(meta_v, [tids, f0 + 5])

        cnt_e1 = plsc.load_gather(cnt0_v, [e1.astype(jnp.int32)])
        pos1 = c1 + cnt_e1
        m0 = pos0 < CAP
        m1 = pos1 < CAP
        q0 = (e0 * CAP + pos0).astype(jnp.int32)
        q1 = (e1 * CAP + pos1).astype(jnp.int32)
        flat0 = jnp.where(m0, rid * ECAP + q0, 0)
        flat1 = jnp.where(m1, rid * ECAP + q1, 0)

        plsc.store_scatter(rows_v, [flat0], g0, mask=m0)
        plsc.store_scatter(rows_v, [flat1], g1, mask=m1)
        pltpu.sync_copy(
            rows_v,
            out_hbm.at[pl.ds((w * TPW + g * GRP) * ECAP, GRP * ECAP)])
        plsc.store_scatter(rows_v, [flat0], zeros16, mask=m0)
        plsc.store_scatter(rows_v, [flat1], zeros16, mask=m1)


_sc_cw = functools.partial(
    pl.kernel,
    out_type=jax.ShapeDtypeStruct((T * ECAP,), jnp.float32),
    mesh=plsc.VectorSubcoreMesh(core_axis_name="c", subcore_axis_name="s"),
    scratch_types=[
        pltpu.VMEM((TPW, 8), jnp.float32),
        pltpu.VMEM((E,), jnp.float32),
        pltpu.VMEM((GRP * ECAP,), jnp.float32),
    ],
    compiler_params=pltpu.CompilerParams(needs_layout_passes=False),
)(_sc_cw_body)


def kernel(x, wg):
    meta, acc = pl.pallas_call(
        _route_kernel,
        grid=(NB,),
        in_specs=[
            pl.BlockSpec((BT, D), lambda i: (i, 0)),
            pl.BlockSpec((D, E), lambda i: (0, 0)),
        ],
        out_shape=[
            jax.ShapeDtypeStruct((NB, BT, 8), jnp.float32),
            jax.ShapeDtypeStruct((8, E), jnp.float32),
        ],
        out_specs=[
            pl.BlockSpec((1, BT, 8), lambda i: (i, 0, 0)),
            pl.BlockSpec((8, E), lambda i: (0, 0)),
        ],
        compiler_params=pltpu.CompilerParams(
            dimension_semantics=("arbitrary",)),
        interpret=False,
    )(x, wg)

    dm = pl.pallas_call(
        _mask_kernel,
        grid=(NB,),
        in_specs=[
            pl.BlockSpec((1, BT, 8), lambda i: (i, 0, 0)),
            pl.BlockSpec((8, E), lambda i: (0, 0)),
        ],
        out_shape=jax.ShapeDtypeStruct((T, ECAP), jnp.int8),
        out_specs=pl.BlockSpec((BT, ECAP), lambda i: (i, 0)),
        compiler_params=pltpu.CompilerParams(
            dimension_semantics=("arbitrary",)),
        interpret=False,
    )(meta, acc)

    cw = _sc_cw(meta, acc)

    l_aux = acc[3, 0]
    exp_counts = acc[0].astype(jnp.int32)
    return (l_aux, cw.reshape(T, E, CAP),
            dm.reshape(T, E, CAP).astype(jnp.bool_), exp_counts)


# speedup vs baseline: 2.3994x; 1.6167x over previous
import functools

import jax
import jax.numpy as jnp
from jax import lax
from jax.experimental import pallas as pl
from jax.experimental.pallas import tpu as pltpu
from jax.experimental.pallas import tpu_sc as plsc

T = 4096
D = 2048
E = 64
CAP = 64
ECAP = E * CAP
THR = 0.4
BT = 256
NB = T // BT

NW = 32
TPW = T // NW
GRP = 16
NGRP = TPW // GRP


def _route_kernel(x_ref, wg_ref, meta_ref, acc_ref):
    i = pl.program_id(0)

    @pl.when(i == 0)
    def _():
        acc_ref[...] = jnp.zeros_like(acc_ref)

    xb = x_ref[...]
    logits = jnp.dot(xb, wg_ref[...], preferred_element_type=jnp.float32)
    mx = jnp.max(logits, axis=1, keepdims=True)
    ex = jnp.exp(logits - mx)
    gates = ex / jnp.sum(ex, axis=1, keepdims=True)

    iota_e = jax.lax.broadcasted_iota(jnp.int32, (BT, E), 1).astype(jnp.float32)
    g0 = jnp.max(gates, axis=1, keepdims=True)
    e0 = jnp.min(jnp.where(gates == g0, iota_e, jnp.float32(E)),
                 axis=1, keepdims=True)
    oh0 = (iota_e == e0).astype(jnp.float32)

    gates1 = jnp.where(iota_e == e0, -jnp.inf, gates)
    g1 = jnp.max(gates1, axis=1, keepdims=True)
    e1 = jnp.min(jnp.where(gates1 == g1, iota_e, jnp.float32(E)),
                 axis=1, keepdims=True)
    flag1 = g0 < THR
    oh1 = ((iota_e == e1) & flag1).astype(jnp.float32)

    rr = jax.lax.broadcasted_iota(jnp.int32, (BT, BT), 0)
    cc = jax.lax.broadcasted_iota(jnp.int32, (BT, BT), 1)
    tril = (rr > cc).astype(jnp.float32)
    pb0 = jnp.dot(tril, oh0, preferred_element_type=jnp.float32)
    pb1 = jnp.dot(tril, oh1, preferred_element_type=jnp.float32)

    cnt0 = acc_ref[0:1, :]
    cnt1 = acc_ref[1:2, :]
    pos0 = jnp.sum((pb0 + cnt0) * oh0, axis=1, keepdims=True)
    c1v = jnp.sum((pb1 + cnt1) * oh1, axis=1, keepdims=True)
    c1 = jnp.where(flag1, c1v, jnp.float32(1e6))

    acc_ref[0:1, :] = cnt0 + jnp.sum(oh0, axis=0, keepdims=True)
    acc_ref[1:2, :] = cnt1 + jnp.sum(oh1, axis=0, keepdims=True)
    acc_ref[2:3, :] = acc_ref[2:3, :] + jnp.sum(gates, axis=0, keepdims=True)

    meta_ref[0] = jnp.concatenate(
        [e0, e1, g0, g1, pos0, c1, jnp.zeros((BT, 2), jnp.float32)], axis=1)

    @pl.when(i == NB - 1)
    def _():
        laux = jnp.sum(acc_ref[0:1, :] * acc_ref[2:3, :]) * (E / (T * T))
        acc_ref[3:4, :] = jnp.full((1, E), laux, jnp.float32)


def _mask_kernel(meta_ref, acc_ref, dm_ref):
    meta = meta_ref[0]
    e0 = meta[:, 0:1]
    e1 = meta[:, 1:2]
    pos0 = meta[:, 4:5]
    c1 = meta[:, 5:6]
    g0 = meta[:, 2:3]
    g1 = meta[:, 3:4]
    cnt0 = acc_ref[0:1, :]

    iota_e = jax.lax.broadcasted_iota(jnp.int32, (BT, E), 1).astype(jnp.float32)
    ohe1 = (iota_e == e1).astype(jnp.float32)
    pos1 = c1 + jnp.sum(cnt0 * ohe1, axis=1, keepdims=True)

    q0 = jnp.where((pos0 < CAP) & (g0 != 0.0),
                   e0 * CAP + pos0, jnp.float32(-1.0))
    q1 = jnp.where((pos1 < CAP) & (g1 != 0.0),
                   e1 * CAP + pos1, jnp.float32(-1.0))

    qi = jax.lax.broadcasted_iota(jnp.int32, (BT, ECAP), 1).astype(jnp.float32)
    dm_ref[...] = ((qi == q0) | (qi == q1)).astype(jnp.int8)


def _sc_cw_body(meta_hbm, acc_hbm, out_hbm, meta_v, cnt0_v, rows_v):
    c = lax.axis_index("c")
    s = lax.axis_index("s")
    w = s * 2 + c

    blk = w // 2
    off = (w % 2) * TPW
    pltpu.sync_copy(meta_hbm.at[blk, pl.ds(off, TPW), :], meta_v)
    pltpu.sync_copy(acc_hbm.at[0], cnt0_v)

    for r in range(GRP):
        def _zrow(k, _, r=r):
            b = k * 64
            z = jnp.zeros((16,), jnp.float32)
            rows_v[r, pl.ds(b, 16)] = z
            rows_v[r, pl.ds(b + 16, 16)] = z
            rows_v[r, pl.ds(b + 32, 16)] = z
            rows_v[r, pl.ds(b + 48, 16)] = z
            return 0
        lax.fori_loop(0, ECAP // 64, _zrow, 0)

    zeros16 = jnp.zeros((16,), jnp.float32)
    rid = lax.iota(jnp.int32, 16)

    for g in range(NGRP):
        tids = rid + g * GRP
        f0 = jnp.full((16,), 0, jnp.int32)
        e0 = plsc.load_gather(meta_v, [tids, f0])
        e1 = plsc.load_gather(meta_v, [tids, f0 + 1])
        g0 = plsc.load_gather(meta_v, [tids, f0 + 2])
        g1 = plsc.load_gather(meta_v, [tids, f0 + 3])
        pos0 = plsc.load_gather(meta_v, [tids, f0 + 4])
        c1 = plsc.load_gather(meta_v, [tids, f0 + 5])

        cnt_e1 = plsc.load_gather(cnt0_v, [e1.astype(jnp.int32)])
        pos1 = c1 + cnt_e1
        m0 = pos0 < CAP
        m1 = pos1 < CAP
        q0 = (e0 * CAP + pos0).astype(jnp.int32)
        q1 = (e1 * CAP + pos1).astype(jnp.int32)
        q0 = jnp.where(m0, q0, 0)
        q1 = jnp.where(m1, q1, 0)

        plsc.store_scatter(rows_v, [rid, q0], g0, mask=m0)
        plsc.store_scatter(rows_v, [rid, q1], g1, mask=m1)
        pltpu.sync_copy(
            rows_v,
            out_hbm.at[pl.ds(w * TPW + g * GRP, GRP), :])
        plsc.store_scatter(rows_v, [rid, q0], zeros16, mask=m0)
        plsc.store_scatter(rows_v, [rid, q1], zeros16, mask=m1)


_sc_cw = functools.partial(
    pl.kernel,
    out_type=jax.ShapeDtypeStruct((T, ECAP), jnp.float32),
    mesh=plsc.VectorSubcoreMesh(core_axis_name="c", subcore_axis_name="s"),
    scratch_types=[
        pltpu.VMEM((TPW, 8), jnp.float32),
        pltpu.VMEM((E,), jnp.float32),
        pltpu.VMEM((GRP, ECAP), jnp.float32),
    ],
    compiler_params=pltpu.CompilerParams(needs_layout_passes=False),
)(_sc_cw_body)


def kernel(x, wg):
    meta, acc = pl.pallas_call(
        _route_kernel,
        grid=(NB,),
        in_specs=[
            pl.BlockSpec((BT, D), lambda i: (i, 0)),
            pl.BlockSpec((D, E), lambda i: (0, 0)),
        ],
        out_shape=[
            jax.ShapeDtypeStruct((NB, BT, 8), jnp.float32),
            jax.ShapeDtypeStruct((8, E), jnp.float32),
        ],
        out_specs=[
            pl.BlockSpec((1, BT, 8), lambda i: (i, 0, 0)),
            pl.BlockSpec((8, E), lambda i: (0, 0)),
        ],
        compiler_params=pltpu.CompilerParams(
            dimension_semantics=("arbitrary",)),
        interpret=False,
    )(x, wg)

    dm = pl.pallas_call(
        _mask_kernel,
        grid=(NB,),
        in_specs=[
            pl.BlockSpec((1, BT, 8), lambda i: (i, 0, 0)),
            pl.BlockSpec((8, E), lambda i: (0, 0)),
        ],
        out_shape=jax.ShapeDtypeStruct((T, ECAP), jnp.int8),
        out_specs=pl.BlockSpec((BT, ECAP), lambda i: (i, 0)),
        compiler_params=pltpu.CompilerParams(
            dimension_semantics=("arbitrary",)),
        interpret=False,
    )(meta, acc)

    cw = _sc_cw(meta, acc)

    l_aux = acc[3, 0]
    exp_counts = acc[0].astype(jnp.int32)
    return (l_aux, cw.reshape(T, E, CAP),
            dm.reshape(T, E, CAP).astype(jnp.bool_), exp_counts)
